# baseline (speedup 1.0000x reference)
import jax
import jax.numpy as jnp
from jax import lax
from jax.experimental import pallas as pl
from jax.experimental.pallas import tpu as pltpu
from jax.experimental.pallas import tpu_sc as plsc

N = 8
TMAX = 512
UMAX = 64
V = 128
NCHUNK = 5
NEG = -1e30
NDIAG = TMAX - 1 + UMAX
HEB = 528
GDP = 80
LUT = 4112
AH = 80 * (NDIAG + 1)


def _log1p_poly(z):
    p = z * (-0.0172077992) + 0.0817256453
    p = p * z + (-0.188780824)
    p = p * z + 0.314589174
    p = p * z + (-0.496977431)
    p = p * z + 0.999792362
    p = p * z + 3.51102136e-06
    return p


_TAKE_DN = lax.GatherDimensionNumbers(
    offset_dims=(), collapsed_slice_dims=(0,), start_index_map=(0,))


def _take(v, idx):
    return lax.gather(v, idx[:, None], _TAKE_DN, slice_sizes=(1,),
                      mode=lax.GatherScatterMode.PROMISE_IN_BOUNDS)


def _body(enc_hbm, dec_hbm, y_hbm, il_hbm, tl_hbm, out_hbm,
          enc_v, dec_v, y_v, il_v, tl_v, hebp, gdp, tnn, ah, out_v, qtab, sem):
    cid = lax.axis_index("c")
    sid = lax.axis_index("s")
    wid = sid * 2 + cid

    @pl.when(wid < N)
    def _run():
        b = wid
        cps = [pltpu.async_copy(enc_hbm.at[b], enc_v, sem),
               pltpu.async_copy(dec_hbm.at[b], dec_v, sem),
               pltpu.async_copy(y_hbm.at[b], y_v, sem),
               pltpu.async_copy(il_hbm, il_v.at[pl.ds(0, N)], sem),
               pltpu.async_copy(tl_hbm, tl_v.at[pl.ds(0, N)], sem)]
        for cp in cps:
            cp.wait()

        iot = lax.iota(jnp.int32, 16)
        zero16 = jnp.zeros((16,), jnp.int32)
        zf16 = jnp.zeros((16,), jnp.float32)
        neg16 = jnp.full((16,), NEG, jnp.float32)
        pm1 = jnp.maximum(iot - 1, 0)
        lane0 = iot == 0
        p15 = jnp.full((16,), 15, jnp.int32)

        b16 = jnp.full((16,), b, jnp.int32)
        t_len = jnp.max(plsc.load_gather(il_v, [b16]))
        u_len = jnp.max(plsc.load_gather(tl_v, [b16]))
        d_target = t_len - 1 + u_len
        fin = (plsc.load_gather(enc_v, [jnp.full((16,), t_len - 1, jnp.int32), zero16])
               + plsc.load_gather(dec_v, [jnp.full((16,), u_len, jnp.int32), zero16]))

        hebp[pl.ds(0, 16)] = zf16
        carr = zf16
        for k in range(TMAX // 16):
            v = plsc.load_gather(enc_v, [iot + 16 * k, zero16])
            cs = carr + plsc.cumsum(v)
            hebp[pl.ds(16 * k + 1, 16)] = cs
            carr = _take(cs, p15)
        gdp[pl.ds(0, 16)] = zf16
        gdp[pl.ds(64, 16)] = zf16
        carr = zf16
        for k in range(UMAX // 16):
            uv = iot + 16 * k
            yv = plsc.load_gather(y_v, [uv])
            v = plsc.load_gather(dec_v, [uv, yv])
            cs = carr + plsc.cumsum(v)
            gdp[pl.ds(16 * k + 1, 16)] = cs
            carr = _take(cs, p15)

        def _build(k, carry):
            delk = ((iot + 16 * k).astype(jnp.float32) + 0.5) * (1.0 / 256.0)
            tnn[pl.ds(k * 16, 16)] = _log1p_poly(jnp.exp(-delk))
            return carry
        lax.fori_loop(0, LUT // 16, _build, jnp.int32(0))

        consts = []
        for c in range(NCHUNK):
            u = iot * NCHUNK + c
            um1 = jnp.clip(u - 1, 0, UMAX - 1)
            yh = plsc.load_gather(y_v, [um1])
            ucl = jnp.minimum(u, UMAX)
            db = plsc.load_gather(dec_v, [ucl, zero16])
            consts.extend((u, yh, db))
        consts = tuple(consts)
        for c in range(NCHUNK):
            plsc.store_scatter(qtab, [iot * NCHUNK + c], iot + (16 * c))

        r_init = [jnp.where(iot == 0, 0.0, NEG)] + [neg16] * (NCHUNK - 1)

        def step(d, rp, cs):
            off = d * 80
            C = range(NCHUNK)
            us = [cs[3 * c] for c in C]
            yhs = [cs[3 * c + 1] for c in C]
            dbs = [cs[3 * c + 2] for c in C]
            tcls = [jnp.clip(d - us[c], 0, TMAX - 1) for c in C]
            ghes = [plsc.load_gather(enc_v, [tcls[c], yhs[c]]) for c in C]
            sh0 = _take(rp[NCHUNK - 1], pm1)
            hps = [jnp.where(lane0, NEG, sh0)] + [rp[c - 1] for c in range(1, NCHUNK)]
            verts = [rp[c] + dbs[c] for c in C]
            horizs = [hps[c] + ghes[c] for c in C]
            ms = [jnp.maximum(verts[c], horizs[c]) for c in C]
            xs = [jnp.minimum(jnp.abs(verts[c] - horizs[c]) * 256.0, 4095.0)
                  for c in C]
            xis = [xs[c].astype(jnp.int32) for c in C]
            gs = [plsc.load_gather(tnn, [xis[c]]) for c in C]
            rn = [ms[c] + gs[c] for c in C]
            for c in C:
                ah[pl.ds(off + 16 * c, 16)] = rn[c]
            return rn

        def loop_body(i, carry):
            cs = carry[:3 * NCHUNK]
            rp = list(carry[3 * NCHUNK:])
            d = 2 * i + 1
            rp = step(d, rp, cs)
            rp = step(d + 1, rp, cs)
            return cs + tuple(rp)

        carry = lax.fori_loop(0, (NDIAG - 1) // 2, loop_body,
                              consts + tuple(r_init))
        step(NDIAG, list(carry[3 * NCHUNK:]), consts)

        slot = plsc.load_gather(qtab, [jnp.full((16,), u_len, jnp.int32)])
        av = plsc.load_gather(ah, [slot + (d_target * 80)])
        he = plsc.load_gather(hebp, [jnp.full((16,), t_len - 1, jnp.int32)])
        gd = plsc.load_gather(gdp, [jnp.full((16,), u_len, jnp.int32)])
        out_v[pl.ds(0, 16)] = av + fin + (he + gd)
        pltpu.sync_copy(out_v, out_hbm.at[b])


@jax.jit
def _rnnt_sc(enc, dec, y, il, tl):
    mesh = plsc.VectorSubcoreMesh(core_axis_name="c", subcore_axis_name="s",
                                  num_cores=2, num_subcores=16)
    f = pl.kernel(
        _body,
        out_type=jax.ShapeDtypeStruct((N, 16), jnp.float32),
        mesh=mesh,
        compiler_params=pltpu.CompilerParams(needs_layout_passes=False),
        scratch_types=[
            pltpu.VMEM((TMAX, V), jnp.float32),
            pltpu.VMEM((UMAX + 1, V), jnp.float32),
            pltpu.VMEM((UMAX,), jnp.int32),
            pltpu.VMEM((16,), jnp.int32),
            pltpu.VMEM((16,), jnp.int32),
            pltpu.VMEM((HEB,), jnp.float32),
            pltpu.VMEM((GDP,), jnp.float32),
            pltpu.VMEM((LUT,), jnp.float32),
            pltpu.VMEM((AH,), jnp.float32),
            pltpu.VMEM((16,), jnp.float32),
            pltpu.VMEM((80,), jnp.int32),
            pltpu.SemaphoreType.DMA,
        ],
    )
    return f(enc, dec, y, il, tl)


def kernel(encoder_out, decoder_out, targets, input_lengths, target_lengths):
    y = targets.astype(jnp.int32)
    il = input_lengths.astype(jnp.int32)
    tl = target_lengths.astype(jnp.int32)
    out = _rnnt_sc(encoder_out, decoder_out, y, il, tl)
    return out[:, 0]


# speedup vs baseline: 1.0290x; 1.0290x over previous
import jax
import jax.numpy as jnp
from jax import lax
from jax.experimental import pallas as pl
from jax.experimental.pallas import tpu as pltpu
from jax.experimental.pallas import tpu_sc as plsc

N = 8
TMAX = 512
UMAX = 64
V = 128
NCHUNK = 5
NEG = -1e30
NDIAG = TMAX - 1 + UMAX
HEB = 528
GDP = 80
LUT = 4112
AH = 80 * (NDIAG + 1)


def _log1p_poly(z):
    p = z * (-0.0172077992) + 0.0817256453
    p = p * z + (-0.188780824)
    p = p * z + 0.314589174
    p = p * z + (-0.496977431)
    p = p * z + 0.999792362
    p = p * z + 3.51102136e-06
    return p


_TAKE_DN = lax.GatherDimensionNumbers(
    offset_dims=(), collapsed_slice_dims=(0,), start_index_map=(0,))


def _take(v, idx):
    return lax.gather(v, idx[:, None], _TAKE_DN, slice_sizes=(1,),
                      mode=lax.GatherScatterMode.PROMISE_IN_BOUNDS)


def _body(enc_hbm, dec_hbm, y_hbm, il_hbm, tl_hbm, out_hbm,
          enc_v, dec_v, y_v, il_v, tl_v, hebp, gdp, tnn, ah, out_v, qtab, sem):
    cid = lax.axis_index("c")
    sid = lax.axis_index("s")
    wid = sid * 2 + cid

    @pl.when(wid < N)
    def _run():
        b = wid
        cps = [pltpu.async_copy(enc_hbm.at[b], enc_v, sem),
               pltpu.async_copy(dec_hbm.at[b], dec_v, sem),
               pltpu.async_copy(y_hbm.at[b], y_v, sem),
               pltpu.async_copy(il_hbm, il_v.at[pl.ds(0, N)], sem),
               pltpu.async_copy(tl_hbm, tl_v.at[pl.ds(0, N)], sem)]

        iot = lax.iota(jnp.int32, 16)
        zero16 = jnp.zeros((16,), jnp.int32)
        zf16 = jnp.zeros((16,), jnp.float32)
        neg16 = jnp.full((16,), NEG, jnp.float32)
        pm1 = jnp.maximum(iot - 1, 0)
        lane0 = iot == 0
        p15 = jnp.full((16,), 15, jnp.int32)

        e0 = jnp.exp(-(iot.astype(jnp.float32) + 0.5) * (1.0 / 256.0))
        rmul = jnp.full((16,), 0.9394130628, jnp.float32)

        def _build(k, e):
            tnn[pl.ds(k * 16, 16)] = _log1p_poly(e)
            return e * rmul
        lax.fori_loop(0, LUT // 16, _build, e0)
        for c in range(NCHUNK):
            plsc.store_scatter(qtab, [iot * NCHUNK + c], iot + (16 * c))

        for cp in cps:
            cp.wait()

        b16 = jnp.full((16,), b, jnp.int32)
        t_len = jnp.max(plsc.load_gather(il_v, [b16]))
        u_len = jnp.max(plsc.load_gather(tl_v, [b16]))
        d_target = t_len - 1 + u_len
        fin = (plsc.load_gather(enc_v, [jnp.full((16,), t_len - 1, jnp.int32), zero16])
               + plsc.load_gather(dec_v, [jnp.full((16,), u_len, jnp.int32), zero16]))

        hebp[pl.ds(0, 16)] = zf16
        carr = zf16
        for k in range(TMAX // 16):
            v = plsc.load_gather(enc_v, [iot + 16 * k, zero16])
            cs = carr + plsc.cumsum(v)
            hebp[pl.ds(16 * k + 1, 16)] = cs
            carr = _take(cs, p15)
        gdp[pl.ds(0, 16)] = zf16
        gdp[pl.ds(64, 16)] = zf16
        carr = zf16
        for k in range(UMAX // 16):
            uv = iot + 16 * k
            yv = plsc.load_gather(y_v, [uv])
            v = plsc.load_gather(dec_v, [uv, yv])
            cs = carr + plsc.cumsum(v)
            gdp[pl.ds(16 * k + 1, 16)] = cs
            carr = _take(cs, p15)


        consts = []
        for c in range(NCHUNK):
            u = iot * NCHUNK + c
            um1 = jnp.clip(u - 1, 0, UMAX - 1)
            yh = plsc.load_gather(y_v, [um1])
            ucl = jnp.minimum(u, UMAX)
            db = plsc.load_gather(dec_v, [ucl, zero16])
            consts.extend((u, yh, db))
        consts = tuple(consts)

        r_init = [jnp.where(iot == 0, 0.0, NEG)] + [neg16] * (NCHUNK - 1)

        def step(d, rp, cs):
            off = d * 80
            C = range(NCHUNK)
            us = [cs[3 * c] for c in C]
            yhs = [cs[3 * c + 1] for c in C]
            dbs = [cs[3 * c + 2] for c in C]
            tcls = [jnp.clip(d - us[c], 0, TMAX - 1) for c in C]
            ghes = [plsc.load_gather(enc_v, [tcls[c], yhs[c]]) for c in C]
            sh0 = _take(rp[NCHUNK - 1], pm1)
            hps = [jnp.where(lane0, NEG, sh0)] + [rp[c - 1] for c in range(1, NCHUNK)]
            verts = [rp[c] + dbs[c] for c in C]
            horizs = [hps[c] + ghes[c] for c in C]
            ms = [jnp.maximum(verts[c], horizs[c]) for c in C]
            ss = [jnp.minimum(verts[c], horizs[c]) for c in C]
            xs = [jnp.minimum((ms[c] - ss[c]) * 256.0, 4095.0) for c in C]
            xis = [xs[c].astype(jnp.int32) for c in C]
            gs = [plsc.load_gather(tnn, [xis[c]]) for c in C]
            rn = [ms[c] + gs[c] for c in C]
            for c in C:
                ah[pl.ds(off + 16 * c, 16)] = rn[c]
            return rn

        def loop_body(i, carry):
            cs = carry[:3 * NCHUNK]
            rp = list(carry[3 * NCHUNK:])
            d = 2 * i + 1
            rp = step(d, rp, cs)
            rp = step(d + 1, rp, cs)
            return cs + tuple(rp)

        carry = lax.fori_loop(0, (NDIAG - 1) // 2, loop_body,
                              consts + tuple(r_init))
        step(NDIAG, list(carry[3 * NCHUNK:]), consts)

        slot = plsc.load_gather(qtab, [jnp.full((16,), u_len, jnp.int32)])
        av = plsc.load_gather(ah, [slot + (d_target * 80)])
        he = plsc.load_gather(hebp, [jnp.full((16,), t_len - 1, jnp.int32)])
        gd = plsc.load_gather(gdp, [jnp.full((16,), u_len, jnp.int32)])
        out_v[pl.ds(0, 16)] = av + fin + (he + gd)
        pltpu.sync_copy(out_v, out_hbm.at[b])


@jax.jit
def _rnnt_sc(enc, dec, y, il, tl):
    mesh = plsc.VectorSubcoreMesh(core_axis_name="c", subcore_axis_name="s",
                                  num_cores=2, num_subcores=16)
    f = pl.kernel(
        _body,
        out_type=jax.ShapeDtypeStruct((N, 16), jnp.float32),
        mesh=mesh,
        compiler_params=pltpu.CompilerParams(needs_layout_passes=False),
        scratch_types=[
            pltpu.VMEM((TMAX, V), jnp.float32),
            pltpu.VMEM((UMAX + 1, V), jnp.float32),
            pltpu.VMEM((UMAX,), jnp.int32),
            pltpu.VMEM((16,), jnp.int32),
            pltpu.VMEM((16,), jnp.int32),
            pltpu.VMEM((HEB,), jnp.float32),
            pltpu.VMEM((GDP,), jnp.float32),
            pltpu.VMEM((LUT,), jnp.float32),
            pltpu.VMEM((AH,), jnp.float32),
            pltpu.VMEM((16,), jnp.float32),
            pltpu.VMEM((80,), jnp.int32),
            pltpu.SemaphoreType.DMA,
        ],
    )
    return f(enc, dec, y, il, tl)


def kernel(encoder_out, decoder_out, targets, input_lengths, target_lengths):
    y = targets.astype(jnp.int32)
    il = input_lengths.astype(jnp.int32)
    tl = target_lengths.astype(jnp.int32)
    out = _rnnt_sc(encoder_out, decoder_out, y, il, tl)
    return out[:, 0]
